# baseline (speedup 1.0000x reference)
import functools

import jax
import jax.numpy as jnp
from jax import lax
from jax.experimental import pallas as pl
from jax.experimental.pallas import tpu as pltpu
from jax.experimental.pallas import tpu_sc as plsc


def _sc_gather(idx_flat, table):
    V, D = table.shape
    N = idx_flat.shape[0]
    info = plsc.get_sparse_core_info()
    NC, NS = info.num_cores, info.num_subcores
    NW = NC * NS
    per_w = N // NW
    CH = 512
    n_ch = per_w // CH
    mesh = plsc.VectorSubcoreMesh(core_axis_name="c", subcore_axis_name="s")

    @functools.partial(
        pl.kernel,
        mesh=mesh,
        out_type=jax.ShapeDtypeStruct((N, D), jnp.float32),
        scratch_types=[
            pltpu.VMEM((CH,), jnp.int32),
            pltpu.VMEM((CH, D), jnp.float32),
            pltpu.SemaphoreType.DMA,
        ],
        compiler_params=pltpu.CompilerParams(use_tc_tiling_on_sc=False),
    )
    def gather_k(idx_hbm, table_hbm, out_hbm, idx_v, rows_v, sem):
        wid = lax.axis_index("s") * NC + lax.axis_index("c")
        base = wid * per_w

        def body(i, carry):
            off = base + i * CH
            pltpu.sync_copy(idx_hbm.at[pl.ds(off, CH)], idx_v)
            pltpu.async_copy(table_hbm.at[idx_v], rows_v, sem).wait()
            pltpu.sync_copy(rows_v, out_hbm.at[pl.ds(off, CH)])
            return carry

        lax.fori_loop(0, n_ch, body, 0)

    return gather_k(idx_flat, table)


def _tc_mlp(emb3, W1, b1, W2, b2):
    B, L, D = emb3.shape
    H = W1.shape[1]
    E = W2.shape[1]
    SB = 256
    grid = (B // SB,)

    def body(emb_ref, w1_ref, b1_ref, w2_ref, b2_ref, out_ref):
        w1 = w1_ref[...]
        acc = jnp.dot(emb_ref[:, 0, :], w1, preferred_element_type=jnp.float32)
        for l in range(1, L):
            acc = jnp.maximum(
                acc,
                jnp.dot(emb_ref[:, l, :], w1, preferred_element_type=jnp.float32),
            )
        h = jnp.maximum(acc + b1_ref[...], 0.0)
        out_ref[...] = (
            jnp.dot(h, w2_ref[...], preferred_element_type=jnp.float32) + b2_ref[...]
        )

    return pl.pallas_call(
        body,
        grid=grid,
        in_specs=[
            pl.BlockSpec((SB, L, D), lambda i: (i, 0, 0)),
            pl.BlockSpec((D, H), lambda i: (0, 0)),
            pl.BlockSpec((1, H), lambda i: (0, 0)),
            pl.BlockSpec((H, E), lambda i: (0, 0)),
            pl.BlockSpec((1, E), lambda i: (0, 0)),
        ],
        out_specs=pl.BlockSpec((SB, E), lambda i: (i, 0)),
        out_shape=jax.ShapeDtypeStruct((B, E), jnp.float32),
    )(emb3, W1, b1.reshape(1, H), W2, b2.reshape(1, E))


def kernel(x, table, W1, b1, W2, b2):
    B, L = x.shape
    V, D = table.shape
    idx = x.reshape(-1).astype(jnp.int32)
    emb = _sc_gather(idx, table)
    return _tc_mlp(emb.reshape(B, L, D), W1, b1, W2, b2)


# speedup vs baseline: 1.0151x; 1.0151x over previous
import functools

import jax
import jax.numpy as jnp
from jax import lax
from jax.experimental import pallas as pl
from jax.experimental.pallas import tpu as pltpu
from jax.experimental.pallas import tpu_sc as plsc


def _sc_gather(idx_flat, table2):
    V2, D2 = table2.shape
    N = idx_flat.shape[0]
    info = plsc.get_sparse_core_info()
    NC, NS = info.num_cores, info.num_subcores
    NW = NC * NS
    per_w = N // NW
    CH = 512
    n_ch = per_w // CH
    mesh = plsc.VectorSubcoreMesh(core_axis_name="c", subcore_axis_name="s")

    @functools.partial(
        pl.kernel,
        mesh=mesh,
        out_type=jax.ShapeDtypeStruct((N, D2), jnp.float32),
        scratch_types=[
            pltpu.VMEM((CH,), jnp.int32),
            pltpu.VMEM((CH, D2), jnp.float32),
            pltpu.SemaphoreType.DMA,
        ],
    )
    def gather_k(idx_hbm, table_hbm, out_hbm, idx_v, rows_v, sem):
        wid = lax.axis_index("s") * NC + lax.axis_index("c")
        base = wid * per_w

        def body(i, carry):
            off = base + i * CH
            pltpu.sync_copy(idx_hbm.at[pl.ds(off, CH)], idx_v)
            pltpu.async_copy(table_hbm.at[idx_v], rows_v, sem).wait()
            pltpu.sync_copy(rows_v, out_hbm.at[pl.ds(off, CH)])
            return carry

        lax.fori_loop(0, n_ch, body, 0)

    return gather_k(idx_flat, table2)


def _tc_mlp(emb3, par, W1, b1, W2, b2):
    B, L, D2 = emb3.shape
    D = D2 // 2
    H = W1.shape[1]
    E = W2.shape[1]
    SB = 256
    grid = (B // SB,)

    def body(emb_ref, par_ref, w1_ref, b1_ref, w2_ref, b2_ref, out_ref):
        w1 = w1_ref[...]
        acc = None
        for l in range(L):
            lo = emb_ref[:, l, :D]
            hi = emb_ref[:, l, D:]
            p = (par_ref[:, l] == 1).reshape(lo.shape[0], 1)
            e = jnp.where(p, hi, lo)
            z = jnp.dot(e, w1, preferred_element_type=jnp.float32)
            acc = z if acc is None else jnp.maximum(acc, z)
        h = jnp.maximum(acc + b1_ref[...], 0.0)
        out_ref[...] = (
            jnp.dot(h, w2_ref[...], preferred_element_type=jnp.float32) + b2_ref[...]
        )

    return pl.pallas_call(
        body,
        grid=grid,
        in_specs=[
            pl.BlockSpec((SB, L, D2), lambda i: (i, 0, 0)),
            pl.BlockSpec((SB, L), lambda i: (i, 0)),
            pl.BlockSpec((D, H), lambda i: (0, 0)),
            pl.BlockSpec((1, H), lambda i: (0, 0)),
            pl.BlockSpec((H, E), lambda i: (0, 0)),
            pl.BlockSpec((1, E), lambda i: (0, 0)),
        ],
        out_specs=pl.BlockSpec((SB, E), lambda i: (i, 0)),
        out_shape=jax.ShapeDtypeStruct((B, E), jnp.float32),
    )(emb3, par, W1, b1.reshape(1, H), W2, b2.reshape(1, E))


def kernel(x, table, W1, b1, W2, b2):
    B, L = x.shape
    V, D = table.shape
    idx = x.reshape(-1).astype(jnp.int32)
    table2 = table.reshape(V // 2, 2 * D)
    emb = _sc_gather(idx >> 1, table2)
    par = (idx & 1).reshape(B, L)
    return _tc_mlp(emb.reshape(B, L, 2 * D), par, W1, b1, W2, b2)
